# Initial kernel scaffold; baseline (speedup 1.0000x reference)
import functools

import jax
import jax.numpy as jnp
from jax import lax
from jax.experimental import pallas as pl
from jax.experimental.pallas import tpu as pltpu
from jax.experimental.pallas import tpu_sc as plsc

N_NODES = 10000
D_IN = 500
N_PAD = 10240
ROWS_PER_SUB = N_PAD // 16
BLK_N = 640
GRID_N = N_PAD // BLK_N
E_REAL = 160000 + N_NODES
NW = 32
EB = 128
EDGES_PER_W = 5376
NBLK = EDGES_PER_W // EB
E_PAD = NW * EDGES_PER_W
PAD_IDX = N_NODES

f32 = jnp.float32


def _take16(vec, idx):
    return lax.gather(
        vec, idx[:, None],
        lax.GatherDimensionNumbers(
            offset_dims=(), collapsed_slice_dims=(0,), start_index_map=(0,)),
        (1,), mode=lax.GatherScatterMode.PROMISE_IN_BOUNDS)


def _tc1_body(x_ref, w1_ref, as_ref, ad_ref, src_ref, dst_ref):
    h = jnp.dot(x_ref[...], w1_ref[...], preferred_element_type=f32)
    a_s = jnp.dot(h, as_ref[...], preferred_element_type=f32)
    a_d = jnp.dot(h, ad_ref[...], preferred_element_type=f32)
    z8 = jnp.zeros((h.shape[0], 8), f32)
    src_ref[...] = jnp.concatenate([h, a_s, z8], axis=1)
    dst_ref[...] = jnp.concatenate([a_d, z8], axis=1)


def _tc1(x_pad, W1, A_s, A_d):
    return pl.pallas_call(
        _tc1_body,
        grid=(GRID_N,),
        in_specs=[
            pl.BlockSpec((BLK_N, D_IN), lambda i: (i, 0)),
            pl.BlockSpec((D_IN, 64), lambda i: (0, 0)),
            pl.BlockSpec((64, 8), lambda i: (0, 0)),
            pl.BlockSpec((64, 8), lambda i: (0, 0)),
        ],
        out_specs=[
            pl.BlockSpec((BLK_N, 80), lambda i: (i, 0)),
            pl.BlockSpec((BLK_N, 16), lambda i: (i, 0)),
        ],
        out_shape=[
            jax.ShapeDtypeStruct((N_PAD, 80), f32),
            jax.ShapeDtypeStruct((N_PAD, 16), f32),
        ],
    )(x_pad, W1, A_s, A_d)


def _tc2_body(pa_ref, pb_ref, s1_ref, s2_ref, b1_ref, w2_ref, ms_ref, md_ref,
              src2_ref, dst2_ref):
    p = pa_ref[...] + pb_ref[...]
    num = jnp.dot(p, s1_ref[...], preferred_element_type=f32)
    den = jnp.dot(p, s2_ref[...], preferred_element_type=f32)
    out1 = num / (den + 1e-16) + b1_ref[...]
    h2 = jnp.dot(out1, w2_ref[...], preferred_element_type=f32)
    lane = lax.broadcasted_iota(jnp.int32, (h2.shape[0], 16), 1)
    one7 = (lane == 7).astype(f32)
    src2_ref[...] = jnp.dot(h2, ms_ref[...], preferred_element_type=f32) + one7
    dst2_ref[...] = jnp.dot(h2, md_ref[...], preferred_element_type=f32)


def _tc2(pa, pb, S1, S2, b1, W2, M_src, M_dst):
    return pl.pallas_call(
        _tc2_body,
        grid=(GRID_N,),
        in_specs=[
            pl.BlockSpec((BLK_N, 80), lambda i: (i, 0)),
            pl.BlockSpec((BLK_N, 80), lambda i: (i, 0)),
            pl.BlockSpec((80, 64), lambda i: (0, 0)),
            pl.BlockSpec((80, 64), lambda i: (0, 0)),
            pl.BlockSpec((1, 64), lambda i: (0, 0)),
            pl.BlockSpec((64, 7), lambda i: (0, 0)),
            pl.BlockSpec((7, 16), lambda i: (0, 0)),
            pl.BlockSpec((7, 16), lambda i: (0, 0)),
        ],
        out_specs=[
            pl.BlockSpec((BLK_N, 16), lambda i: (i, 0)),
            pl.BlockSpec((BLK_N, 16), lambda i: (i, 0)),
        ],
        out_shape=[
            jax.ShapeDtypeStruct((N_PAD, 16), f32),
            jax.ShapeDtypeStruct((N_PAD, 16), f32),
        ],
    )(pa, pb, S1, S2, b1, W2, M_src, M_dst)


def _tc3_body(pa_ref, pb_ref, t1_ref, t2_ref, b2_ref, out_ref):
    p = pa_ref[...] + pb_ref[...]
    num = jnp.dot(p, t1_ref[...], preferred_element_type=f32)
    den = jnp.dot(p, t2_ref[...], preferred_element_type=f32)
    o = num / (den + 1e-16) + b2_ref[...]
    m = jnp.max(o, axis=1, keepdims=True)
    e = jnp.exp(o - m)
    out_ref[...] = (o - m) - jnp.log(jnp.sum(e, axis=1, keepdims=True))


def _tc3(pa, pb, T1, T2, b2):
    return pl.pallas_call(
        _tc3_body,
        grid=(GRID_N,),
        in_specs=[
            pl.BlockSpec((BLK_N, 16), lambda i: (i, 0)),
            pl.BlockSpec((BLK_N, 16), lambda i: (i, 0)),
            pl.BlockSpec((16, 7), lambda i: (0, 0)),
            pl.BlockSpec((16, 7), lambda i: (0, 0)),
            pl.BlockSpec((1, 7), lambda i: (0, 0)),
        ],
        out_specs=pl.BlockSpec((BLK_N, 7), lambda i: (i, 0)),
        out_shape=jax.ShapeDtypeStruct((N_PAD, 7), f32),
    )(pa, pb, T1, T2, b2)


def _sc_edge_pass(row_w):
    mesh = plsc.VectorSubcoreMesh(core_axis_name="c", subcore_axis_name="s")

    @functools.partial(
        pl.kernel, mesh=mesh,
        out_type=jax.ShapeDtypeStruct((2, N_PAD, row_w), f32),
        scratch_types=[
            pltpu.VMEM((EB,), jnp.int32),
            pltpu.VMEM((EB,), jnp.int32),
            pltpu.VMEM((EB, row_w), f32),
            pltpu.VMEM((EB, 16), f32),
            pltpu.VMEM((EB, row_w), f32),
            pltpu.VMEM_SHARED((N_PAD, row_w), f32),
            pltpu.SemaphoreType.DMA,
        ],
    )
    def k(srci, dsti, stab, dtab, zer, out, sidx, didx, srows, drows, msg,
          acc, sem):
        c = lax.axis_index("c")
        s = lax.axis_index("s")
        w = c * 16 + s
        pltpu.sync_copy(zer.at[pl.ds(s * ROWS_PER_SUB, ROWS_PER_SUB)],
                        acc.at[pl.ds(s * ROWS_PER_SUB, ROWS_PER_SUB)])
        plsc.subcore_barrier()

        lane = lax.iota(jnp.int32, 16)
        mask01 = (lane < 8).astype(f32)
        base0 = w * EDGES_PER_W

        def blk_body(i, carry):
            base = base0 + i * EB
            pltpu.sync_copy(srci.at[pl.ds(base, EB)], sidx)
            pltpu.sync_copy(dsti.at[pl.ds(base, EB)], didx)
            pltpu.async_copy(stab.at[sidx], srows, sem).wait()
            pltpu.async_copy(dtab.at[didx], drows, sem).wait()

            if row_w == 80:
                def edge_body(b, cc):
                    alpha = srows[b, 64:80] + drows[b, :]
                    lr = jnp.where(alpha > 0, alpha, alpha * 0.2)
                    ex = jnp.exp(lr)
                    msg[b, 64:80] = ex * mask01
                    for kk in range(4):
                        bidx = jnp.where(lane < 8, 2 * kk, 2 * kk + 1)
                        exbr = _take16(ex, bidx)
                        msg[b, 16 * kk:16 * (kk + 1)] = (
                            srows[b, 16 * kk:16 * (kk + 1)] * exbr)
                    return cc
            else:
                def edge_body(b, cc):
                    t = srows[b, :] + drows[b, :]
                    lr = jnp.where(t > 0, t, t * 0.2)
                    ex = jnp.exp(lr)
                    exbr = _take16(ex, jnp.full((16,), 8, jnp.int32))
                    msg[b, :] = exbr * srows[b, :] * mask01
                    return cc

            lax.fori_loop(0, EB, edge_body, 0)
            pltpu.sync_copy(msg, acc.at[didx], add=True)
            return carry

        lax.fori_loop(0, NBLK, blk_body, 0)
        plsc.subcore_barrier()

        @pl.when(s == 0)
        def _():
            pltpu.sync_copy(acc, out.at[c])

    return k


_sc_pass_80 = _sc_edge_pass(80)
_sc_pass_16 = _sc_edge_pass(16)


def kernel(x, edge_index, W1, att_src1, att_dst1, bias1, W2, att_src2,
           att_dst2, bias2):
    loop = jnp.arange(N_NODES, dtype=jnp.int32)
    src = jnp.concatenate([edge_index[0].astype(jnp.int32), loop,
                           jnp.full((E_PAD - E_REAL,), PAD_IDX, jnp.int32)])
    dst = jnp.concatenate([edge_index[1].astype(jnp.int32), loop,
                           jnp.full((E_PAD - E_REAL,), PAD_IDX, jnp.int32)])
    x_pad = jnp.zeros((N_PAD, D_IN), f32).at[:N_NODES].set(x.astype(f32))

    eye8 = jnp.eye(8, dtype=f32)
    A_s = (att_src1[0].astype(f32)[:, :, None] * eye8[:, None, :]).reshape(64, 8)
    A_d = (att_dst1[0].astype(f32)[:, :, None] * eye8[:, None, :]).reshape(64, 8)

    S1 = jnp.zeros((80, 64), f32).at[:64, :].set(jnp.eye(64, dtype=f32))
    S2 = jnp.zeros((80, 64), f32).at[64:72, :].set(
        jnp.repeat(eye8, 8, axis=1))
    v_s2 = att_src2[0, 0].astype(f32)
    v_d2 = att_dst2[0, 0].astype(f32)
    M_src = jnp.zeros((7, 16), f32).at[:, :7].set(jnp.eye(7, dtype=f32)) \
        .at[:, 8].set(v_s2)
    M_dst = jnp.zeros((7, 16), f32).at[:, 8].set(v_d2)
    T1 = jnp.zeros((16, 7), f32).at[:7, :].set(jnp.eye(7, dtype=f32))
    T2 = jnp.zeros((16, 7), f32).at[7, :].set(1.0)

    zeros80 = jnp.zeros((N_PAD, 80), f32)
    zeros16 = jnp.zeros((N_PAD, 16), f32)
    b1 = bias1.astype(f32).reshape(1, 64)
    b2 = bias2.astype(f32).reshape(1, 7)

    src_tab, dst_tab = _tc1(x_pad, W1.astype(f32), A_s, A_d)
    acc1 = _sc_pass_80(src, dst, src_tab, dst_tab, zeros80)
    src2, dst2 = _tc2(acc1[0], acc1[1], S1, S2, b1, W2.astype(f32),
                      M_src, M_dst)
    acc2 = _sc_pass_16(src, dst, src2, dst2, zeros16)
    out = _tc3(acc2[0], acc2[1], T1, T2, b2)
    return out[:N_NODES]


# speedup vs baseline: 43.0269x; 43.0269x over previous
import functools

import jax
import jax.numpy as jnp
from jax import lax
from jax.experimental import pallas as pl
from jax.experimental.pallas import tpu as pltpu
from jax.experimental.pallas import tpu_sc as plsc

N_NODES = 10000
D_IN = 500
N_PAD = 10240
BLK_N = 640
GRID_N = N_PAD // BLK_N
E_REAL = 160000 + N_NODES
E_PAD = 172032
CE = 2688
GPC = CE // 16
PAD_IDX = N_NODES

f32 = jnp.float32


def _tc1_body(xT_ref, w1T_ref, asT_ref, adT_ref, hT_ref, aso_ref, ado_ref):
    hT = jnp.dot(w1T_ref[...], xT_ref[...], preferred_element_type=f32)
    hT_ref[...] = hT
    aso_ref[...] = jnp.dot(asT_ref[...], hT, preferred_element_type=f32)
    ado_ref[...] = jnp.dot(adT_ref[...], hT, preferred_element_type=f32)


def _tc1(xT, W1T, A_sT, A_dT):
    return pl.pallas_call(
        _tc1_body,
        grid=(GRID_N,),
        in_specs=[
            pl.BlockSpec((D_IN, BLK_N), lambda i: (0, i)),
            pl.BlockSpec((64, D_IN), lambda i: (0, 0)),
            pl.BlockSpec((8, 64), lambda i: (0, 0)),
            pl.BlockSpec((8, 64), lambda i: (0, 0)),
        ],
        out_specs=[
            pl.BlockSpec((64, BLK_N), lambda i: (0, i)),
            pl.BlockSpec((8, BLK_N), lambda i: (0, i)),
            pl.BlockSpec((8, BLK_N), lambda i: (0, i)),
        ],
        out_shape=[
            jax.ShapeDtypeStruct((64, N_PAD), f32),
            jax.ShapeDtypeStruct((8, N_PAD), f32),
            jax.ShapeDtypeStruct((8, N_PAD), f32),
        ],
    )(xT, W1T, A_sT, A_dT)


def _tc2_body(dp_ref, num_ref, q_ref, r8_ref, b1_ref, w2T_ref, vs_ref, vd_ref,
              p78_ref, h2p_ref, as2_ref, ad2_ref):
    den8 = jnp.dot(q_ref[...], dp_ref[...], preferred_element_type=f32)
    denr = jnp.dot(r8_ref[...], den8, preferred_element_type=f32)
    out1T = num_ref[...] / (denr + 1e-16) + b1_ref[...]
    h2T = jnp.dot(w2T_ref[...], out1T, preferred_element_type=f32)
    h2p_ref[...] = jnp.dot(p78_ref[...], h2T, preferred_element_type=f32)
    as2_ref[...] = jnp.dot(vs_ref[...], h2T, preferred_element_type=f32)
    ad2_ref[...] = jnp.dot(vd_ref[...], h2T, preferred_element_type=f32)


def _tc2(denp1, numT, Q, R8, b1c, W2T, vs2, vd2, P78):
    return pl.pallas_call(
        _tc2_body,
        grid=(GRID_N,),
        in_specs=[
            pl.BlockSpec((32, BLK_N), lambda i: (0, i)),
            pl.BlockSpec((64, BLK_N), lambda i: (0, i)),
            pl.BlockSpec((8, 32), lambda i: (0, 0)),
            pl.BlockSpec((64, 8), lambda i: (0, 0)),
            pl.BlockSpec((64, 1), lambda i: (0, 0)),
            pl.BlockSpec((7, 64), lambda i: (0, 0)),
            pl.BlockSpec((1, 7), lambda i: (0, 0)),
            pl.BlockSpec((1, 7), lambda i: (0, 0)),
            pl.BlockSpec((8, 7), lambda i: (0, 0)),
        ],
        out_specs=[
            pl.BlockSpec((8, BLK_N), lambda i: (0, i)),
            pl.BlockSpec((1, BLK_N), lambda i: (0, i)),
            pl.BlockSpec((1, BLK_N), lambda i: (0, i)),
        ],
        out_shape=[
            jax.ShapeDtypeStruct((8, N_PAD), f32),
            jax.ShapeDtypeStruct((1, N_PAD), f32),
            jax.ShapeDtypeStruct((1, N_PAD), f32),
        ],
    )(denp1, numT, Q, R8, b1c, W2T, vs2, vd2, P78)


def _tc3_body(dp_ref, np_ref, ones_ref, b2_ref, out_ref):
    den = jnp.dot(ones_ref[...], dp_ref[...], preferred_element_type=f32)
    nb = np_ref[0] + np_ref[1] + np_ref[2] + np_ref[3]
    o = nb[0:7, :] / (den + 1e-16) + b2_ref[...]
    m = jnp.max(o, axis=0, keepdims=True)
    e = jnp.exp(o - m)
    out_ref[...] = (o - m) - jnp.log(jnp.sum(e, axis=0, keepdims=True))


def _tc3(denp2, nump2, ones32, b2c):
    return pl.pallas_call(
        _tc3_body,
        grid=(GRID_N,),
        in_specs=[
            pl.BlockSpec((32, BLK_N), lambda i: (0, i)),
            pl.BlockSpec((4, 8, BLK_N), lambda i: (0, 0, i)),
            pl.BlockSpec((1, 32), lambda i: (0, 0)),
            pl.BlockSpec((7, 1), lambda i: (0, 0)),
        ],
        out_specs=pl.BlockSpec((7, BLK_N), lambda i: (0, i)),
        out_shape=jax.ShapeDtypeStruct((7, N_PAD), f32),
    )(denp2, nump2, ones32, b2c)


def _sc_weights(nh):
    nq = 32 // nh
    eq = E_PAD // nq
    nchunk = eq // CE
    mesh = plsc.VectorSubcoreMesh(core_axis_name="c", subcore_axis_name="s",
                                  num_cores=2)

    @functools.partial(
        pl.kernel, mesh=mesh,
        compiler_params=pltpu.CompilerParams(needs_layout_passes=False),
        out_type=[
            jax.ShapeDtypeStruct((nh, E_PAD), f32),
            jax.ShapeDtypeStruct((32, N_PAD), f32),
        ],
        scratch_types=[
            pltpu.VMEM((CE,), jnp.int32),
            pltpu.VMEM((CE,), jnp.int32),
            pltpu.VMEM((CE,), f32),
            pltpu.VMEM((N_PAD,), f32),
            pltpu.VMEM((N_PAD,), f32),
            pltpu.VMEM((N_PAD,), f32),
        ],
    )
    def k(srci, dsti, asT, adT, exo, denp, sidx, didx, exb, asb, adb, den):
        t = lax.axis_index("c") * 16 + lax.axis_index("s")
        hd = t % nh
        q = t // nh

        pltpu.sync_copy(asT.at[hd], asb)
        pltpu.sync_copy(adT.at[hd], adb)

        z16 = jnp.zeros((16,), f32)
        lane = lax.iota(jnp.int32, 16)
        def zrow(i, cc):
            plsc.store_scatter(den, [lane + i * 16], z16)
            return cc
        lax.fori_loop(0, N_PAD // 16, zrow, 0)

        def chunk(ch, cc):
            base = q * eq + ch * CE
            pltpu.sync_copy(srci.at[pl.ds(base, CE)], sidx)
            pltpu.sync_copy(dsti.at[pl.ds(base, CE)], didx)

            def grp(g, gg):
                ev = lane + g * 16
                s16 = plsc.load_gather(sidx, [ev])
                d16 = plsc.load_gather(didx, [ev])
                av = plsc.load_gather(asb, [s16])
                bv = plsc.load_gather(adb, [d16])
                al = av + bv
                lr = jnp.where(al > 0, al, al * 0.2)
                ex = jnp.exp(lr)
                plsc.store_scatter(exb, [ev], ex)
                plsc.addupdate_scatter(den, [d16], ex)
                return gg
            lax.fori_loop(0, GPC, grp, 0)
            pltpu.sync_copy(exb, exo.at[hd].at[pl.ds(base, CE)])
            return cc
        lax.fori_loop(0, nchunk, chunk, 0)

        pltpu.sync_copy(den, denp.at[t])

    return k


def _sc_numerator(n_rows, rpt):
    ntc = n_rows // rpt
    nq = 32 // ntc
    eq = E_PAD // nq
    nchunk = eq // CE
    mesh = plsc.VectorSubcoreMesh(core_axis_name="c", subcore_axis_name="s",
                                  num_cores=2)

    def make(nh_ex):
        rows_per_head = n_rows // nh_ex

        @functools.partial(
            pl.kernel, mesh=mesh,
            compiler_params=pltpu.CompilerParams(needs_layout_passes=False),
            out_type=jax.ShapeDtypeStruct((nq, n_rows, N_PAD), f32),
            scratch_types=[
                pltpu.VMEM((CE,), jnp.int32),
                pltpu.VMEM((CE,), jnp.int32),
                pltpu.VMEM((CE,), f32),
                pltpu.VMEM((rpt, N_PAD), f32),
                pltpu.VMEM((rpt, N_PAD), f32),
            ],
        )
        def k(srci, dsti, htab, exo, numo, sidx, didx, exb, hbuf, acc):
            t = lax.axis_index("c") * 16 + lax.axis_index("s")
            c0 = (t % ntc) * rpt
            q = t // ntc
            hd = c0 // rows_per_head

            pltpu.sync_copy(htab.at[pl.ds(c0, rpt)], hbuf)

            z16 = jnp.zeros((16,), f32)
            lane = lax.iota(jnp.int32, 16)
            jfs = [jnp.full((16,), j, jnp.int32) for j in range(rpt)]
            def zrow(i, cc):
                for j in range(rpt):
                    plsc.store_scatter(acc, [jfs[j], lane + i * 16], z16)
                return cc
            lax.fori_loop(0, N_PAD // 16, zrow, 0)

            def chunk(ch, cc):
                base = q * eq + ch * CE
                pltpu.sync_copy(srci.at[pl.ds(base, CE)], sidx)
                pltpu.sync_copy(dsti.at[pl.ds(base, CE)], didx)
                pltpu.sync_copy(exo.at[hd].at[pl.ds(base, CE)], exb)

                def grp(g, gg):
                    ev = lane + g * 16
                    s16 = plsc.load_gather(sidx, [ev])
                    d16 = plsc.load_gather(didx, [ev])
                    ex = plsc.load_gather(exb, [ev])
                    for j in range(rpt):
                        hv = plsc.load_gather(hbuf, [jfs[j], s16])
                        plsc.addupdate_scatter(acc, [jfs[j], d16], hv * ex)
                    return gg
                lax.fori_loop(0, GPC, grp, 0)
                return cc
            lax.fori_loop(0, nchunk, chunk, 0)

            pltpu.sync_copy(acc, numo.at[q].at[pl.ds(c0, rpt)])

        return k
    return make


_sc_w1 = _sc_weights(8)
_sc_w2 = _sc_weights(1)
_sc_n1 = _sc_numerator(64, 2)(8)
_sc_n2 = _sc_numerator(8, 1)(1)


def kernel(x, edge_index, W1, att_src1, att_dst1, bias1, W2, att_src2,
           att_dst2, bias2):
    loop = jnp.arange(N_NODES, dtype=jnp.int32)
    pad = jnp.full((E_PAD - E_REAL,), PAD_IDX, jnp.int32)
    src = jnp.concatenate([edge_index[0].astype(jnp.int32), loop, pad])
    dst = jnp.concatenate([edge_index[1].astype(jnp.int32), loop, pad])
    xT = jnp.zeros((D_IN, N_PAD), f32).at[:, :N_NODES].set(x.astype(f32).T)

    eye8 = jnp.eye(8, dtype=f32)
    A_sT = (eye8[:, :, None] * att_src1[0].astype(f32)[:, None, :]).reshape(8, 64)
    A_dT = (eye8[:, :, None] * att_dst1[0].astype(f32)[:, None, :]).reshape(8, 64)
    Q = jnp.concatenate([eye8, eye8, eye8, eye8], axis=1)
    R8 = jnp.repeat(eye8, 8, axis=0)
    P78 = jnp.eye(8, 7, dtype=f32)
    ones32 = jnp.ones((1, 32), f32)
    vs2 = att_src2[0].astype(f32)
    vd2 = att_dst2[0].astype(f32)
    b1c = bias1.astype(f32).reshape(64, 1)
    b2c = bias2.astype(f32).reshape(7, 1)

    hT, asT, adT = _tc1(xT, W1.astype(f32).T, A_sT, A_dT)
    ex1, denp1 = _sc_w1(src, dst, asT, adT)
    num1 = _sc_n1(src, dst, hT, ex1)
    h2pT, as2T, ad2T = _tc2(denp1, num1[0], Q, R8, b1c, W2.astype(f32).T,
                            vs2, vd2, P78)
    ex2, denp2 = _sc_w2(src, dst, as2T, ad2T)
    nump2 = _sc_n2(src, dst, h2pT, ex2)
    outT = _tc3(denp2, nump2, ones32, b2c)
    return outT.T[:N_NODES]


# speedup vs baseline: 53.8810x; 1.2523x over previous
import functools

import jax
import jax.numpy as jnp
from jax import lax
from jax.experimental import pallas as pl
from jax.experimental.pallas import tpu as pltpu
from jax.experimental.pallas import tpu_sc as plsc

N_NODES = 10000
D_IN = 500
N_PAD = 10240
BLK_N = 640
GRID_N = N_PAD // BLK_N
E_REAL = 160000 + N_NODES
E_PAD = 172032
CE = 2688
GPC = CE // 16
PAD_IDX = N_NODES

f32 = jnp.float32


def _tc1_body(xT_ref, w1T_ref, asT_ref, adT_ref, hT_ref, aso_ref, ado_ref):
    hT = jnp.dot(w1T_ref[...], xT_ref[...], preferred_element_type=f32)
    hT_ref[...] = hT
    aso_ref[...] = jnp.dot(asT_ref[...], hT, preferred_element_type=f32)
    ado_ref[...] = jnp.dot(adT_ref[...], hT, preferred_element_type=f32)


def _tc1(xT, W1T, A_sT, A_dT):
    return pl.pallas_call(
        _tc1_body,
        grid=(GRID_N,),
        in_specs=[
            pl.BlockSpec((D_IN, BLK_N), lambda i: (0, i)),
            pl.BlockSpec((64, D_IN), lambda i: (0, 0)),
            pl.BlockSpec((8, 64), lambda i: (0, 0)),
            pl.BlockSpec((8, 64), lambda i: (0, 0)),
        ],
        out_specs=[
            pl.BlockSpec((64, BLK_N), lambda i: (0, i)),
            pl.BlockSpec((8, BLK_N), lambda i: (0, i)),
            pl.BlockSpec((8, BLK_N), lambda i: (0, i)),
        ],
        out_shape=[
            jax.ShapeDtypeStruct((64, N_PAD), f32),
            jax.ShapeDtypeStruct((8, N_PAD), f32),
            jax.ShapeDtypeStruct((8, N_PAD), f32),
        ],
    )(xT, W1T, A_sT, A_dT)


def _tc2_body(dp_ref, numa_ref, numb_ref, q_ref, r8_ref, b1_ref, w2T_ref,
              vs_ref, vd_ref, p78_ref, h2p_ref, as2_ref, ad2_ref):
    den8 = jnp.dot(q_ref[...], dp_ref[...], preferred_element_type=f32)
    denr = jnp.dot(r8_ref[...], den8, preferred_element_type=f32)
    out1T = (numa_ref[...] + numb_ref[...]) / (denr + 1e-16) + b1_ref[...]
    h2T = jnp.dot(w2T_ref[...], out1T, preferred_element_type=f32)
    h2p_ref[...] = jnp.dot(p78_ref[...], h2T, preferred_element_type=f32)
    as2_ref[...] = jnp.dot(vs_ref[...], h2T, preferred_element_type=f32)
    ad2_ref[...] = jnp.dot(vd_ref[...], h2T, preferred_element_type=f32)


def _tc2(denp1, numa, numb, Q, R8, b1c, W2T, vs2, vd2, P78):
    return pl.pallas_call(
        _tc2_body,
        grid=(GRID_N,),
        in_specs=[
            pl.BlockSpec((32, BLK_N), lambda i: (0, i)),
            pl.BlockSpec((64, BLK_N), lambda i: (0, i)),
            pl.BlockSpec((64, BLK_N), lambda i: (0, i)),
            pl.BlockSpec((8, 32), lambda i: (0, 0)),
            pl.BlockSpec((64, 8), lambda i: (0, 0)),
            pl.BlockSpec((64, 1), lambda i: (0, 0)),
            pl.BlockSpec((7, 64), lambda i: (0, 0)),
            pl.BlockSpec((1, 7), lambda i: (0, 0)),
            pl.BlockSpec((1, 7), lambda i: (0, 0)),
            pl.BlockSpec((8, 7), lambda i: (0, 0)),
        ],
        out_specs=[
            pl.BlockSpec((8, BLK_N), lambda i: (0, i)),
            pl.BlockSpec((1, BLK_N), lambda i: (0, i)),
            pl.BlockSpec((1, BLK_N), lambda i: (0, i)),
        ],
        out_shape=[
            jax.ShapeDtypeStruct((8, N_PAD), f32),
            jax.ShapeDtypeStruct((1, N_PAD), f32),
            jax.ShapeDtypeStruct((1, N_PAD), f32),
        ],
    )(denp1, numa, numb, Q, R8, b1c, W2T, vs2, vd2, P78)


def _tc3_body(dp_ref, np_ref, ones_ref, b2_ref, out_ref):
    den = jnp.dot(ones_ref[...], dp_ref[...], preferred_element_type=f32)
    nb = np_ref[0]
    for _qq in range(1, 16):
        nb = nb + np_ref[_qq]
    o = nb[0:7, :] / (den + 1e-16) + b2_ref[...]
    m = jnp.max(o, axis=0, keepdims=True)
    e = jnp.exp(o - m)
    out_ref[...] = (o - m) - jnp.log(jnp.sum(e, axis=0, keepdims=True))


def _tc3(denp2, nump2, ones32, b2c):
    return pl.pallas_call(
        _tc3_body,
        grid=(GRID_N,),
        in_specs=[
            pl.BlockSpec((32, BLK_N), lambda i: (0, i)),
            pl.BlockSpec((16, 8, BLK_N), lambda i: (0, 0, i)),
            pl.BlockSpec((1, 32), lambda i: (0, 0)),
            pl.BlockSpec((7, 1), lambda i: (0, 0)),
        ],
        out_specs=pl.BlockSpec((7, BLK_N), lambda i: (0, i)),
        out_shape=jax.ShapeDtypeStruct((7, N_PAD), f32),
    )(denp2, nump2, ones32, b2c)


def _sc_weights(nh):
    nq = 32 // nh
    eq = E_PAD // nq
    nchunk = eq // CE
    mesh = plsc.VectorSubcoreMesh(core_axis_name="c", subcore_axis_name="s",
                                  num_cores=2)

    @functools.partial(
        pl.kernel, mesh=mesh,
        compiler_params=pltpu.CompilerParams(needs_layout_passes=False),
        out_type=[
            jax.ShapeDtypeStruct((nh, E_PAD), f32),
            jax.ShapeDtypeStruct((32, N_PAD), f32),
        ],
        scratch_types=[
            pltpu.VMEM((CE,), jnp.int32),
            pltpu.VMEM((CE,), jnp.int32),
            pltpu.VMEM((CE,), f32),
            pltpu.VMEM((N_PAD,), f32),
            pltpu.VMEM((N_PAD,), f32),
            pltpu.VMEM((N_PAD,), f32),
        ],
    )
    def k(srci, dsti, asT, adT, exo, denp, sidx, didx, exb, asb, adb, den):
        t = lax.axis_index("c") * 16 + lax.axis_index("s")
        hd = t % nh
        q = t // nh

        pltpu.sync_copy(asT.at[hd], asb)
        pltpu.sync_copy(adT.at[hd], adb)

        z16 = jnp.zeros((16,), f32)
        lane = lax.iota(jnp.int32, 16)
        def zrow(i, cc):
            plsc.store_scatter(den, [lane + i * 16], z16)
            return cc
        lax.fori_loop(0, N_PAD // 16, zrow, 0)

        def chunk(ch, cc):
            base = q * eq + ch * CE
            pltpu.sync_copy(srci.at[pl.ds(base, CE)], sidx)
            pltpu.sync_copy(dsti.at[pl.ds(base, CE)], didx)

            def grp(g, gg):
                o = g * 16
                s16 = sidx[pl.ds(o, 16)]
                d16 = didx[pl.ds(o, 16)]
                av = plsc.load_gather(asb, [s16])
                bv = plsc.load_gather(adb, [d16])
                al = av + bv
                lr = jnp.where(al > 0, al, al * 0.2)
                ex = jnp.exp(lr)
                exb[pl.ds(o, 16)] = ex
                plsc.addupdate_scatter(den, [d16], ex)
                return gg
            lax.fori_loop(0, GPC, grp, 0, unroll=8)
            pltpu.sync_copy(exb, exo.at[hd].at[pl.ds(base, CE)])
            return cc
        lax.fori_loop(0, nchunk, chunk, 0)

        pltpu.sync_copy(den, denp.at[t])

    return k


def _sc_numerator(n_rows, rpt):
    ntc = n_rows // rpt
    nq = 32 // ntc
    eq = E_PAD // nq
    nchunk = eq // CE
    mesh = plsc.VectorSubcoreMesh(core_axis_name="c", subcore_axis_name="s",
                                  num_cores=2)

    def make(nh_ex):
        rows_per_head = n_rows // nh_ex

        @functools.partial(
            pl.kernel, mesh=mesh,
            compiler_params=pltpu.CompilerParams(needs_layout_passes=False),
            out_type=jax.ShapeDtypeStruct((nq, n_rows, N_PAD), f32),
            scratch_types=[
                pltpu.VMEM((CE,), jnp.int32),
                pltpu.VMEM((CE,), jnp.int32),
                pltpu.VMEM((CE,), f32),
                pltpu.VMEM((rpt, N_PAD), f32),
                pltpu.VMEM((rpt, N_PAD), f32),
            ],
        )
        def k(srci, dsti, htab, exo, numo, sidx, didx, exb, hbuf, acc):
            t = lax.axis_index("c") * 16 + lax.axis_index("s")
            c0 = (t % ntc) * rpt
            q = t // ntc
            hd = c0 // rows_per_head

            pltpu.sync_copy(htab.at[pl.ds(c0, rpt)], hbuf)

            z16 = jnp.zeros((16,), f32)
            lane = lax.iota(jnp.int32, 16)
            jfs = [jnp.full((16,), j, jnp.int32) for j in range(rpt)]
            def zrow(i, cc):
                for j in range(rpt):
                    plsc.store_scatter(acc, [jfs[j], lane + i * 16], z16)
                return cc
            lax.fori_loop(0, N_PAD // 16, zrow, 0)

            def chunk(ch, cc):
                base = q * eq + ch * CE
                pltpu.sync_copy(srci.at[pl.ds(base, CE)], sidx)
                pltpu.sync_copy(dsti.at[pl.ds(base, CE)], didx)
                pltpu.sync_copy(exo.at[hd].at[pl.ds(base, CE)], exb)

                def grp(g, gg):
                    o = g * 16
                    s16 = sidx[pl.ds(o, 16)]
                    d16 = didx[pl.ds(o, 16)]
                    ex = exb[pl.ds(o, 16)]
                    for j in range(rpt):
                        hv = plsc.load_gather(hbuf, [jfs[j], s16])
                        plsc.addupdate_scatter(acc, [jfs[j], d16], hv * ex)
                    return gg
                lax.fori_loop(0, GPC, grp, 0, unroll=4)
                return cc
            lax.fori_loop(0, nchunk, chunk, 0)

            pltpu.sync_copy(acc, numo.at[q].at[pl.ds(c0, rpt)])

        return k
    return make


_sc_w1 = _sc_weights(8)
_sc_w2 = _sc_weights(1)
_sc_n1 = _sc_numerator(64, 4)(8)
_sc_n2 = _sc_numerator(8, 4)(1)


def kernel(x, edge_index, W1, att_src1, att_dst1, bias1, W2, att_src2,
           att_dst2, bias2):
    loop = jnp.arange(N_NODES, dtype=jnp.int32)
    pad = jnp.full((E_PAD - E_REAL,), PAD_IDX, jnp.int32)
    src = jnp.concatenate([edge_index[0].astype(jnp.int32), loop, pad])
    dst = jnp.concatenate([edge_index[1].astype(jnp.int32), loop, pad])
    xT = jnp.zeros((D_IN, N_PAD), f32).at[:, :N_NODES].set(x.astype(f32).T)

    eye8 = jnp.eye(8, dtype=f32)
    A_sT = (eye8[:, :, None] * att_src1[0].astype(f32)[:, None, :]).reshape(8, 64)
    A_dT = (eye8[:, :, None] * att_dst1[0].astype(f32)[:, None, :]).reshape(8, 64)
    Q = jnp.concatenate([eye8, eye8, eye8, eye8], axis=1)
    R8 = jnp.repeat(eye8, 8, axis=0)
    P78 = jnp.eye(8, 7, dtype=f32)
    ones32 = jnp.ones((1, 32), f32)
    vs2 = att_src2[0].astype(f32)
    vd2 = att_dst2[0].astype(f32)
    b1c = bias1.astype(f32).reshape(64, 1)
    b2c = bias2.astype(f32).reshape(7, 1)

    hT, asT, adT = _tc1(xT, W1.astype(f32).T, A_sT, A_dT)
    ex1, denp1 = _sc_w1(src, dst, asT, adT)
    num1p = _sc_n1(src, dst, hT, ex1)
    h2pT, as2T, ad2T = _tc2(denp1, num1p[0], num1p[1], Q, R8, b1c, W2.astype(f32).T,
                            vs2, vd2, P78)
    ex2, denp2 = _sc_w2(src, dst, as2T, ad2T)
    nump2 = _sc_n2(src, dst, h2pT, ex2)
    outT = _tc3(denp2, nump2, ones32, b2c)
    return outT.T[:N_NODES]
